# baseline (speedup 1.0000x reference)
import functools

import jax
import jax.numpy as jnp
import numpy as np
from jax import lax
from jax.experimental import pallas as pl
from jax.experimental.pallas import tpu as pltpu
from jax.experimental.pallas import tpu_sc as plsc

_ACT_CST = 1.6790

_NC = 2
_NS = 16
_LANES = 16


def _wf_tc(edge_embed_t, edge_feat_t, W0, W1, W2p, eoff, epart):
    R, E = edge_embed_t.shape
    H = W0.shape[1]
    D = W2p.shape[1]
    BE = 6400
    assert epart % BE == 0 and eoff % BE == 0
    s0 = 1.0 / np.sqrt(W0.shape[0])
    s1 = 1.0 / np.sqrt(W1.shape[0])
    cdims = (((0,), (0,)), ((), ()))

    def body(ee_ref, ef_ref, w0_ref, w1_ref, w2_ref, out_ref):
        x = ee_ref[...]
        h = lax.dot_general(w0_ref[...] * s0, x, cdims,
                            preferred_element_type=jnp.float32)
        h = jax.nn.silu(h) * _ACT_CST
        h = lax.dot_general(w1_ref[...] * s1, h, cdims,
                            preferred_element_type=jnp.float32)
        h = jax.nn.silu(h) * _ACT_CST
        h = h * ef_ref[...]
        wf = lax.dot_general(h, w2_ref[...], cdims,
                             preferred_element_type=jnp.float32)
        out_ref[...] = pltpu.bitcast(wf.astype(jnp.bfloat16), jnp.int32)

    ob = eoff // BE
    return pl.pallas_call(
        body,
        grid=(epart // BE,),
        in_specs=[
            pl.BlockSpec((R, BE), lambda i: (0, i + ob)),
            pl.BlockSpec((1, BE), lambda i: (0, i + ob)),
            pl.BlockSpec((W0.shape[0], H), lambda i: (0, 0)),
            pl.BlockSpec((H, H), lambda i: (0, 0)),
            pl.BlockSpec((H, D), lambda i: (0, 0)),
        ],
        out_specs=pl.BlockSpec((BE // 2, D), lambda i: (i, 0)),
        out_shape=jax.ShapeDtypeStruct((epart // 2, D), jnp.int32),
    )(edge_embed_t, edge_feat_t, W0, W1, W2p)


def _gather_scatter_sc(src, dst, wf, node_feat, zeros, K, eoff, epart):
    NW = _NC * _NS
    CH = epart // (NW * K)
    KH = K // 2
    N, D = node_feat.shape
    RPT = (N // _NS) & ~7
    TAIL = N - RPT * _NS

    mesh = plsc.VectorSubcoreMesh(core_axis_name="c", subcore_axis_name="s")

    @functools.partial(
        pl.kernel,
        out_type=[jax.ShapeDtypeStruct((N, D), jnp.float32)] * 2,
        mesh=mesh,
        scratch_types=[
            [pltpu.VMEM((K,), jnp.int32)] * 4,
            [pltpu.VMEM((K,), jnp.int32)] * 4,
            [pltpu.VMEM((K, D), jnp.float32)] * 2,
            [pltpu.VMEM((KH, D), jnp.int32)] * 2,
            pltpu.VMEM_SHARED((N, D), jnp.float32),
            [pltpu.SemaphoreType.DMA] * 4,
            [pltpu.SemaphoreType.DMA] * 2,
            [pltpu.SemaphoreType.DMA] * 2,
        ],
    )
    def sc_kernel(src_hbm, dst_hbm, wf_hbm, node_hbm, zeros_hbm,
                  out0, out1, srcb, dstb, rows, wfb, acc,
                  isem, gsem, wsem):
        c = lax.axis_index("c")
        s = lax.axis_index("s")
        wid = c * _NS + s
        base0h = wid * (CH * KH)

        pltpu.sync_copy(zeros_hbm.at[pl.ds(0, RPT)],
                        acc.at[pl.ds(s * RPT, RPT)])
        if TAIL:
            @pl.when(s == 0)
            def _():
                pltpu.sync_copy(zeros_hbm.at[pl.ds(0, TAIL)],
                                acc.at[pl.ds(RPT * _NS, TAIL)])
        plsc.subcore_barrier()

        def _idx_copies(i, q):
            base = eoff + (wid * CH + i) * K
            return (
                pltpu.make_async_copy(src_hbm.at[pl.ds(base, K)],
                                      srcb[q], isem[q]),
                pltpu.make_async_copy(dst_hbm.at[pl.ds(base, K)],
                                      dstb[q], isem[q]),
            )

        def start_idx(i, q):
            for cp in _idx_copies(i, q):
                cp.start()

        def wait_idx(i, q):
            for cp in _idx_copies(i, q):
                cp.wait()

        def start_data(i, p, q):
            pltpu.async_copy(node_hbm.at[srcb[q]], rows[p], gsem[p])
            pltpu.async_copy(wf_hbm.at[pl.ds(base0h + i * KH, KH)],
                             wfb[p], wsem[p])

        def wait_data(i, p, q):
            pltpu.make_async_copy(node_hbm.at[srcb[q]], rows[p],
                                  gsem[p]).wait()
            pltpu.make_async_copy(wf_hbm.at[pl.ds(base0h + i * KH, KH)],
                                  wfb[p], wsem[p]).wait()

        for q in range(4):
            start_idx(q, q)
        for p in range(2):
            wait_idx(p, p)
            start_data(p, p, p)

        def body(i, p, q):
            q2 = (q + 2) % 4
            wait_data(i, p, q)

            wfb_bf = wfb[p].bitcast(jnp.bfloat16).reshape(KH, 2, D)

            @plsc.parallel_loop(0, KH, unroll=2)
            def _(pp):
                for sub in range(2):
                    e = pp * 2 + sub
                    for m in range(D // (4 * _LANES)):
                        sl = pl.ds(m * 4 * _LANES, 4 * _LANES)
                        w32 = wfb_bf[pp, sub, sl].astype(jnp.float32)
                        rows[p][e, sl] = rows[p][e, sl] * w32

            pltpu.sync_copy(rows[p], acc.at[dstb[q]], add=True)

            @pl.when(i + 2 < CH)
            def _():
                wait_idx(i + 2, q2)
                start_data(i + 2, p, q2)

            @pl.when(i + 4 < CH)
            def _():
                start_idx(i + 4, q)

        def chunk(i, carry):
            for p in range(2):
                for q in (p, p + 2):
                    @pl.when(lax.rem(i, 4) == q)
                    def _(i=i, p=p, q=q):
                        body(i, p, q)
            return carry

        lax.fori_loop(0, CH, chunk, 0)
        plsc.subcore_barrier()

        def drain(out):
            pltpu.sync_copy(acc.at[pl.ds(s * RPT, RPT)],
                            out.at[pl.ds(s * RPT, RPT)])
            if TAIL:
                @pl.when(s == 0)
                def _():
                    pltpu.sync_copy(acc.at[pl.ds(RPT * _NS, TAIL)],
                                    out.at[pl.ds(RPT * _NS, TAIL)])

        @pl.when(c == 0)
        def _():
            drain(out0)

        @pl.when(c == 1)
        def _():
            drain(out1)

    return sc_kernel(src, dst, wf, node_feat, zeros)


def _combine_tc(partials, node_feat, W_sc):
    N, D = node_feat.shape
    BN = 2000
    assert N % BN == 0
    ssc = 1.0 / np.sqrt(W_sc.shape[0])
    NP = len(partials)

    def body(*refs):
        *p_refs, nf_ref, wsc_ref, out_ref = refs
        acc = jnp.dot(nf_ref[...], wsc_ref[...] * ssc,
                      preferred_element_type=jnp.float32)
        for pr in p_refs:
            acc = acc + pr[...]
        out_ref[...] = acc

    return pl.pallas_call(
        body,
        grid=(N // BN,),
        in_specs=[pl.BlockSpec((BN, D), lambda i: (i, 0))] * (NP + 1)
        + [pl.BlockSpec((D, D), lambda i: (0, 0))],
        out_specs=pl.BlockSpec((BN, D), lambda i: (i, 0)),
        out_shape=jax.ShapeDtypeStruct((N, D), jnp.float32),
    )(*partials, node_feat, W_sc)


def kernel(edge_index, node_feat, edge_feat, edge_embed, dim_size,
           W0, W1, W2, W_sc):
    N, D = node_feat.shape
    E = edge_index.shape[1]
    W2p = W2 * (1.0 / np.sqrt(W2.shape[0]))
    zeros = jnp.zeros((N // _NS, D), jnp.float32)
    src, dst = edge_index[0], edge_index[1]
    ee_t, ef_t = edge_embed.T, edge_feat.T
    ea, eb = 128000, 102400
    parts = [(0, ea), (ea, eb), (ea + eb, E - ea - eb)]
    partials = []
    for eoff, epart in parts:
        wf = _wf_tc(ee_t, ef_t, W0, W1, W2p, eoff, epart)
        partials += _gather_scatter_sc(src, dst, wf, node_feat, zeros,
                                       K=80, eoff=eoff, epart=epart)
    return _combine_tc(partials, node_feat, W_sc)


# speedup vs baseline: 1.0416x; 1.0416x over previous
import functools

import jax
import jax.numpy as jnp
import numpy as np
from jax import lax
from jax.experimental import pallas as pl
from jax.experimental.pallas import tpu as pltpu
from jax.experimental.pallas import tpu_sc as plsc

_ACT_CST = 1.6790

_NC = 2
_NS = 16
_LANES = 16


def _wf_tc(edge_embed_t, edge_feat_t, W0, W1, W2p, eoff, epart):
    R, E = edge_embed_t.shape
    H = W0.shape[1]
    D = W2p.shape[1]
    BE = 12800
    assert epart % BE == 0 and eoff % BE == 0
    s0 = 1.0 / np.sqrt(W0.shape[0])
    s1 = 1.0 / np.sqrt(W1.shape[0])
    cdims = (((0,), (0,)), ((), ()))

    def body(ee_ref, ef_ref, w0_ref, w1_ref, w2_ref, out_ref):
        x = ee_ref[...]
        h = lax.dot_general(w0_ref[...] * s0, x, cdims,
                            preferred_element_type=jnp.float32)
        h = jax.nn.silu(h) * _ACT_CST
        h = lax.dot_general(w1_ref[...] * s1, h, cdims,
                            preferred_element_type=jnp.float32)
        h = jax.nn.silu(h) * _ACT_CST
        h = h * ef_ref[...]
        wf = lax.dot_general(h, w2_ref[...], cdims,
                             preferred_element_type=jnp.float32)
        out_ref[...] = pltpu.bitcast(wf.astype(jnp.bfloat16), jnp.int32)

    ob = eoff // BE
    return pl.pallas_call(
        body,
        grid=(epart // BE,),
        in_specs=[
            pl.BlockSpec((R, BE), lambda i: (0, i + ob)),
            pl.BlockSpec((1, BE), lambda i: (0, i + ob)),
            pl.BlockSpec((W0.shape[0], H), lambda i: (0, 0)),
            pl.BlockSpec((H, H), lambda i: (0, 0)),
            pl.BlockSpec((H, D), lambda i: (0, 0)),
        ],
        out_specs=pl.BlockSpec((BE // 2, D), lambda i: (i, 0)),
        out_shape=jax.ShapeDtypeStruct((epart // 2, D), jnp.int32),
    )(edge_embed_t, edge_feat_t, W0, W1, W2p)


def _gather_scatter_sc(src, dst, wf, node_feat, zeros, K, eoff, epart):
    NW = _NC * _NS
    CH = epart // (NW * K)
    KH = K // 2
    N, D = node_feat.shape
    RPT = (N // _NS) & ~7
    TAIL = N - RPT * _NS

    mesh = plsc.VectorSubcoreMesh(core_axis_name="c", subcore_axis_name="s")

    @functools.partial(
        pl.kernel,
        out_type=[jax.ShapeDtypeStruct((N, D), jnp.float32)] * 2,
        mesh=mesh,
        scratch_types=[
            [pltpu.VMEM((K,), jnp.int32)] * 4,
            [pltpu.VMEM((K,), jnp.int32)] * 4,
            [pltpu.VMEM((K, D), jnp.float32)] * 2,
            [pltpu.VMEM((KH, D), jnp.int32)] * 2,
            pltpu.VMEM_SHARED((N, D), jnp.float32),
            [pltpu.SemaphoreType.DMA] * 4,
            [pltpu.SemaphoreType.DMA] * 2,
            [pltpu.SemaphoreType.DMA] * 2,
        ],
    )
    def sc_kernel(src_hbm, dst_hbm, wf_hbm, node_hbm, zeros_hbm,
                  out0, out1, srcb, dstb, rows, wfb, acc,
                  isem, gsem, wsem):
        c = lax.axis_index("c")
        s = lax.axis_index("s")
        wid = c * _NS + s
        base0h = wid * (CH * KH)

        pltpu.sync_copy(zeros_hbm.at[pl.ds(0, RPT)],
                        acc.at[pl.ds(s * RPT, RPT)])
        if TAIL:
            @pl.when(s == 0)
            def _():
                pltpu.sync_copy(zeros_hbm.at[pl.ds(0, TAIL)],
                                acc.at[pl.ds(RPT * _NS, TAIL)])
        plsc.subcore_barrier()

        def _idx_copies(i, q):
            base = eoff + (wid * CH + i) * K
            return (
                pltpu.make_async_copy(src_hbm.at[pl.ds(base, K)],
                                      srcb[q], isem[q]),
                pltpu.make_async_copy(dst_hbm.at[pl.ds(base, K)],
                                      dstb[q], isem[q]),
            )

        def start_idx(i, q):
            for cp in _idx_copies(i, q):
                cp.start()

        def wait_idx(i, q):
            for cp in _idx_copies(i, q):
                cp.wait()

        def start_data(i, p, q):
            pltpu.async_copy(node_hbm.at[srcb[q]], rows[p], gsem[p])
            pltpu.async_copy(wf_hbm.at[pl.ds(base0h + i * KH, KH)],
                             wfb[p], wsem[p])

        def wait_data(i, p, q):
            pltpu.make_async_copy(node_hbm.at[srcb[q]], rows[p],
                                  gsem[p]).wait()
            pltpu.make_async_copy(wf_hbm.at[pl.ds(base0h + i * KH, KH)],
                                  wfb[p], wsem[p]).wait()

        for q in range(4):
            start_idx(q, q)
        for p in range(2):
            wait_idx(p, p)
            start_data(p, p, p)

        def body(i, p, q):
            q2 = (q + 2) % 4
            wait_data(i, p, q)

            wfb_bf = wfb[p].bitcast(jnp.bfloat16).reshape(KH, 2, D)

            @plsc.parallel_loop(0, KH, unroll=2)
            def _(pp):
                for sub in range(2):
                    e = pp * 2 + sub
                    for m in range(D // (4 * _LANES)):
                        sl = pl.ds(m * 4 * _LANES, 4 * _LANES)
                        w32 = wfb_bf[pp, sub, sl].astype(jnp.float32)
                        rows[p][e, sl] = rows[p][e, sl] * w32

            pltpu.sync_copy(rows[p], acc.at[dstb[q]], add=True)

            @pl.when(i + 2 < CH)
            def _():
                wait_idx(i + 2, q2)
                start_data(i + 2, p, q2)

            @pl.when(i + 4 < CH)
            def _():
                start_idx(i + 4, q)

        def chunk(i, carry):
            for p in range(2):
                for q in (p, p + 2):
                    @pl.when(lax.rem(i, 4) == q)
                    def _(i=i, p=p, q=q):
                        body(i, p, q)
            return carry

        lax.fori_loop(0, CH, chunk, 0)
        plsc.subcore_barrier()

        def drain(out):
            pltpu.sync_copy(acc.at[pl.ds(s * RPT, RPT)],
                            out.at[pl.ds(s * RPT, RPT)])
            if TAIL:
                @pl.when(s == 0)
                def _():
                    pltpu.sync_copy(acc.at[pl.ds(RPT * _NS, TAIL)],
                                    out.at[pl.ds(RPT * _NS, TAIL)])

        @pl.when(c == 0)
        def _():
            drain(out0)

        @pl.when(c == 1)
        def _():
            drain(out1)

    return sc_kernel(src, dst, wf, node_feat, zeros)


def _combine_tc(partials, node_feat, W_sc):
    N, D = node_feat.shape
    BN = 2000
    assert N % BN == 0
    ssc = 1.0 / np.sqrt(W_sc.shape[0])
    NP = len(partials)

    def body(*refs):
        *p_refs, nf_ref, wsc_ref, out_ref = refs
        acc = jnp.dot(nf_ref[...], wsc_ref[...] * ssc,
                      preferred_element_type=jnp.float32)
        for pr in p_refs:
            acc = acc + pr[...]
        out_ref[...] = acc

    return pl.pallas_call(
        body,
        grid=(N // BN,),
        in_specs=[pl.BlockSpec((BN, D), lambda i: (i, 0))] * (NP + 1)
        + [pl.BlockSpec((D, D), lambda i: (0, 0))],
        out_specs=pl.BlockSpec((BN, D), lambda i: (i, 0)),
        out_shape=jax.ShapeDtypeStruct((N, D), jnp.float32),
    )(*partials, node_feat, W_sc)


def kernel(edge_index, node_feat, edge_feat, edge_embed, dim_size,
           W0, W1, W2, W_sc):
    N, D = node_feat.shape
    E = edge_index.shape[1]
    W2p = W2 * (1.0 / np.sqrt(W2.shape[0]))
    zeros = jnp.zeros((N // _NS, D), jnp.float32)
    src, dst = edge_index[0], edge_index[1]
    ee_t, ef_t = edge_embed.T, edge_feat.T
    ea = (E * 3 // 5 // 12800) * 12800
    parts = [(0, ea), (ea, E - ea)]
    partials = []
    for eoff, epart in parts:
        wf = _wf_tc(ee_t, ef_t, W0, W1, W2p, eoff, epart)
        partials += _gather_scatter_sc(src, dst, wf, node_feat, zeros,
                                       K=80, eoff=eoff, epart=epart)
    return _combine_tc(partials, node_feat, W_sc)


# speedup vs baseline: 1.0528x; 1.0107x over previous
import functools

import jax
import jax.numpy as jnp
import numpy as np
from jax import lax
from jax.experimental import pallas as pl
from jax.experimental.pallas import tpu as pltpu
from jax.experimental.pallas import tpu_sc as plsc

_ACT_CST = 1.6790

_NC = 2
_NS = 16
_LANES = 16


def _wf_tc(edge_embed_t, edge_feat_t, W0, W1, W2p, eoff, epart):
    R, E = edge_embed_t.shape
    H = W0.shape[1]
    D = W2p.shape[1]
    BE = 16000
    assert epart % BE == 0 and eoff % BE == 0
    s0 = 1.0 / np.sqrt(W0.shape[0])
    s1 = 1.0 / np.sqrt(W1.shape[0])
    cdims = (((0,), (0,)), ((), ()))

    def body(ee_ref, ef_ref, w0_ref, w1_ref, w2_ref, out_ref):
        x = ee_ref[...]
        h = lax.dot_general(w0_ref[...] * s0, x, cdims,
                            preferred_element_type=jnp.float32)
        h = jax.nn.silu(h) * _ACT_CST
        h = lax.dot_general(w1_ref[...] * s1, h, cdims,
                            preferred_element_type=jnp.float32)
        h = jax.nn.silu(h) * _ACT_CST
        h = h * ef_ref[...]
        wf = lax.dot_general(h, w2_ref[...], cdims,
                             preferred_element_type=jnp.float32)
        out_ref[...] = pltpu.bitcast(wf.astype(jnp.bfloat16), jnp.int32)

    ob = eoff // BE
    return pl.pallas_call(
        body,
        grid=(epart // BE,),
        in_specs=[
            pl.BlockSpec((R, BE), lambda i: (0, i + ob)),
            pl.BlockSpec((1, BE), lambda i: (0, i + ob)),
            pl.BlockSpec((W0.shape[0], H), lambda i: (0, 0)),
            pl.BlockSpec((H, H), lambda i: (0, 0)),
            pl.BlockSpec((H, D), lambda i: (0, 0)),
        ],
        out_specs=pl.BlockSpec((BE // 2, D), lambda i: (i, 0)),
        out_shape=jax.ShapeDtypeStruct((epart // 2, D), jnp.int32),
    )(edge_embed_t, edge_feat_t, W0, W1, W2p)


def _gather_scatter_sc(src, dst, wf, node_feat, zeros, K, eoff, epart):
    NW = _NC * _NS
    CH = epart // (NW * K)
    KH = K // 2
    N, D = node_feat.shape
    RPT = (N // _NS) & ~7
    TAIL = N - RPT * _NS

    mesh = plsc.VectorSubcoreMesh(core_axis_name="c", subcore_axis_name="s")

    @functools.partial(
        pl.kernel,
        out_type=[jax.ShapeDtypeStruct((N, D), jnp.float32)] * 2,
        mesh=mesh,
        scratch_types=[
            [pltpu.VMEM((K,), jnp.int32)] * 4,
            [pltpu.VMEM((K,), jnp.int32)] * 4,
            [pltpu.VMEM((K, D), jnp.float32)] * 2,
            [pltpu.VMEM((KH, D), jnp.int32)] * 2,
            pltpu.VMEM_SHARED((N, D), jnp.float32),
            [pltpu.SemaphoreType.DMA] * 4,
            [pltpu.SemaphoreType.DMA] * 2,
            [pltpu.SemaphoreType.DMA] * 2,
        ],
    )
    def sc_kernel(src_hbm, dst_hbm, wf_hbm, node_hbm, zeros_hbm,
                  out0, out1, srcb, dstb, rows, wfb, acc,
                  isem, gsem, wsem):
        c = lax.axis_index("c")
        s = lax.axis_index("s")
        wid = c * _NS + s
        base0h = wid * (CH * KH)

        pltpu.sync_copy(zeros_hbm.at[pl.ds(0, RPT)],
                        acc.at[pl.ds(s * RPT, RPT)])
        if TAIL:
            @pl.when(s == 0)
            def _():
                pltpu.sync_copy(zeros_hbm.at[pl.ds(0, TAIL)],
                                acc.at[pl.ds(RPT * _NS, TAIL)])
        plsc.subcore_barrier()

        def _idx_copies(i, q):
            base = eoff + (wid * CH + i) * K
            return (
                pltpu.make_async_copy(src_hbm.at[pl.ds(base, K)],
                                      srcb[q], isem[q]),
                pltpu.make_async_copy(dst_hbm.at[pl.ds(base, K)],
                                      dstb[q], isem[q]),
            )

        def start_idx(i, q):
            for cp in _idx_copies(i, q):
                cp.start()

        def wait_idx(i, q):
            for cp in _idx_copies(i, q):
                cp.wait()

        def start_data(i, p, q):
            pltpu.async_copy(node_hbm.at[srcb[q]], rows[p], gsem[p])
            pltpu.async_copy(wf_hbm.at[pl.ds(base0h + i * KH, KH)],
                             wfb[p], wsem[p])

        def wait_data(i, p, q):
            pltpu.make_async_copy(node_hbm.at[srcb[q]], rows[p],
                                  gsem[p]).wait()
            pltpu.make_async_copy(wf_hbm.at[pl.ds(base0h + i * KH, KH)],
                                  wfb[p], wsem[p]).wait()

        for q in range(4):
            start_idx(q, q)
        for p in range(2):
            wait_idx(p, p)
            start_data(p, p, p)

        def body(i, p, q):
            q2 = (q + 2) % 4
            wait_data(i, p, q)

            wfb_bf = wfb[p].bitcast(jnp.bfloat16).reshape(KH, 2, D)

            @plsc.parallel_loop(0, KH, unroll=2)
            def _(pp):
                for sub in range(2):
                    e = pp * 2 + sub
                    for m in range(D // (4 * _LANES)):
                        sl = pl.ds(m * 4 * _LANES, 4 * _LANES)
                        w32 = wfb_bf[pp, sub, sl].astype(jnp.float32)
                        rows[p][e, sl] = rows[p][e, sl] * w32

            pltpu.sync_copy(rows[p], acc.at[dstb[q]], add=True)

            @pl.when(i + 2 < CH)
            def _():
                wait_idx(i + 2, q2)
                start_data(i + 2, p, q2)

            @pl.when(i + 4 < CH)
            def _():
                start_idx(i + 4, q)

        def chunk(i, carry):
            for p in range(2):
                for q in (p, p + 2):
                    @pl.when(lax.rem(i, 4) == q)
                    def _(i=i, p=p, q=q):
                        body(i, p, q)
            return carry

        lax.fori_loop(0, CH, chunk, 0)
        plsc.subcore_barrier()

        def drain(out):
            pltpu.sync_copy(acc.at[pl.ds(s * RPT, RPT)],
                            out.at[pl.ds(s * RPT, RPT)])
            if TAIL:
                @pl.when(s == 0)
                def _():
                    pltpu.sync_copy(acc.at[pl.ds(RPT * _NS, TAIL)],
                                    out.at[pl.ds(RPT * _NS, TAIL)])

        @pl.when(c == 0)
        def _():
            drain(out0)

        @pl.when(c == 1)
        def _():
            drain(out1)

    return sc_kernel(src, dst, wf, node_feat, zeros)


def _combine_tc(partials, node_feat, W_sc):
    N, D = node_feat.shape
    BN = 2000
    assert N % BN == 0
    ssc = 1.0 / np.sqrt(W_sc.shape[0])
    NP = len(partials)

    def body(*refs):
        *p_refs, nf_ref, wsc_ref, out_ref = refs
        acc = jnp.dot(nf_ref[...], wsc_ref[...] * ssc,
                      preferred_element_type=jnp.float32)
        for pr in p_refs:
            acc = acc + pr[...]
        out_ref[...] = acc

    return pl.pallas_call(
        body,
        grid=(N // BN,),
        in_specs=[pl.BlockSpec((BN, D), lambda i: (i, 0))] * (NP + 1)
        + [pl.BlockSpec((D, D), lambda i: (0, 0))],
        out_specs=pl.BlockSpec((BN, D), lambda i: (i, 0)),
        out_shape=jax.ShapeDtypeStruct((N, D), jnp.float32),
    )(*partials, node_feat, W_sc)


def kernel(edge_index, node_feat, edge_feat, edge_embed, dim_size,
           W0, W1, W2, W_sc):
    N, D = node_feat.shape
    E = edge_index.shape[1]
    W2p = W2 * (1.0 / np.sqrt(W2.shape[0]))
    zeros = jnp.zeros((N // _NS, D), jnp.float32)
    src, dst = edge_index[0], edge_index[1]
    ee_t, ef_t = edge_embed.T, edge_feat.T
    ea = (E * 3 // 5 // 12800) * 12800
    parts = [(0, ea), (ea, E - ea)]
    partials = []
    for eoff, epart in parts:
        wf = _wf_tc(ee_t, ef_t, W0, W1, W2p, eoff, epart)
        partials += _gather_scatter_sc(src, dst, wf, node_feat, zeros,
                                       K=80, eoff=eoff, epart=epart)
    return _combine_tc(partials, node_feat, W_sc)
